# baseline (speedup 1.0000x reference)
import functools

import jax
import jax.numpy as jnp
from jax import lax
from jax.experimental import pallas as pl
from jax.experimental.pallas import tpu as pltpu
from jax.experimental.pallas import tpu_sc as plsc

N_VOCABS = 1000000
EMB_DIM = 64
BATCH = 4096
SEQLEN = 200

CHUNK = 128
TOTAL = BATCH * SEQLEN
NROWS = TOTAL // CHUNK
NW = 32
ROWS_PER_W = NROWS // NW


def _emb_body(idx_hbm, table_hbm, out_hbm, idx_v, rows_v, gsem, osem):
    cid = lax.axis_index("c")
    sid = lax.axis_index("s")
    wid = sid * 2 + cid
    base = wid * ROWS_PER_W

    pltpu.sync_copy(idx_hbm.at[pl.ds(base, ROWS_PER_W)], idx_v)

    @pl.loop(0, ROWS_PER_W)
    def _chunk(j):
        pltpu.async_copy(table_hbm.at[idx_v.at[j]], rows_v, gsem).wait()
        pltpu.async_copy(rows_v, out_hbm.at[base + j], osem).wait()


@functools.partial(jax.jit, static_argnames=())
def _emb_lookup(idx, table):
    k = pl.kernel(
        _emb_body,
        out_type=jax.ShapeDtypeStruct((NROWS, CHUNK, EMB_DIM), jnp.float32),
        mesh=plsc.VectorSubcoreMesh(core_axis_name="c", subcore_axis_name="s"),
        compiler_params=pltpu.CompilerParams(use_tc_tiling_on_sc=False),
        scratch_types=[
            pltpu.VMEM((ROWS_PER_W, CHUNK), jnp.int32),
            pltpu.VMEM((CHUNK, EMB_DIM), jnp.float32),
            pltpu.SemaphoreType.DMA,
            pltpu.SemaphoreType.DMA,
        ],
    )
    return k(idx, table)


def kernel(input, emb_weight):
    idx = input.reshape(NROWS, CHUNK)
    out = _emb_lookup(idx, emb_weight)
    return out.reshape(BATCH, SEQLEN, EMB_DIM)


# speedup vs baseline: 1.1186x; 1.1186x over previous
import functools

import jax
import jax.numpy as jnp
from jax import lax
from jax.experimental import pallas as pl
from jax.experimental.pallas import tpu as pltpu
from jax.experimental.pallas import tpu_sc as plsc

N_VOCABS = 1000000
EMB_DIM = 64
BATCH = 4096
SEQLEN = 200

CHUNK = 128
TOTAL = BATCH * SEQLEN
NROWS = TOTAL // CHUNK
NW = 32
ROWS_PER_W = NROWS // NW
NBUF = 8
G = 4


def _emb_body(idx_hbm, table_hbm, out_hbm, idx_v, rows_v, *sems):
    gsem = sems[:NBUF]
    osem = sems[NBUF:]
    cid = lax.axis_index("c")
    sid = lax.axis_index("s")
    wid = sid * 2 + cid
    base = wid * ROWS_PER_W

    pltpu.sync_copy(idx_hbm.at[pl.ds(base, ROWS_PER_W)], idx_v)

    def start_gather(j, slot):
        pltpu.make_async_copy(
            table_hbm.at[idx_v.at[j]], rows_v.at[slot], gsem[slot]
        ).start()

    def wait_gather(j, slot):
        pltpu.make_async_copy(
            table_hbm.at[idx_v.at[j]], rows_v.at[slot], gsem[slot]
        ).wait()

    def start_write(j, slot):
        pltpu.make_async_copy(
            rows_v.at[slot], out_hbm.at[base + j], osem[slot]
        ).start()

    def wait_write(j, slot):
        pltpu.make_async_copy(
            rows_v.at[slot], out_hbm.at[base + j], osem[slot]
        ).wait()

    for j in range(G):
        start_gather(j, j % NBUF)
    for j in range(NBUF - G):
        wait_gather(j, j % NBUF)
        start_write(j, j % NBUF)
        start_gather(j + G, (j + G) % NBUF)

    @pl.loop(NBUF - G, ROWS_PER_W - G, step=NBUF)
    def _steady(j0):
        for b in range(NBUF):
            j = j0 + b
            slot = (NBUF - G + b) % NBUF
            sn = (NBUF - G + b + G) % NBUF
            wait_gather(j, slot)
            start_write(j, slot)
            wait_write(j + G - NBUF, sn)
            start_gather(j + G, sn)

    for j in range(ROWS_PER_W - G, ROWS_PER_W):
        wait_gather(j, j % NBUF)
        start_write(j, j % NBUF)
    for j in range(ROWS_PER_W - NBUF, ROWS_PER_W):
        wait_write(j, j % NBUF)


@jax.jit
def _emb_lookup(idx, table):
    k = pl.kernel(
        _emb_body,
        out_type=jax.ShapeDtypeStruct((NROWS, CHUNK, EMB_DIM), jnp.float32),
        mesh=plsc.VectorSubcoreMesh(core_axis_name="c", subcore_axis_name="s"),
        compiler_params=pltpu.CompilerParams(use_tc_tiling_on_sc=False),
        scratch_types=(
            [
                pltpu.VMEM((ROWS_PER_W, CHUNK), jnp.int32),
                pltpu.VMEM((NBUF, CHUNK, EMB_DIM), jnp.float32),
            ]
            + [pltpu.SemaphoreType.DMA] * (2 * NBUF)
        ),
    )
    return k(idx, table)


def kernel(input, emb_weight):
    idx = input.reshape(NROWS, CHUNK)
    out = _emb_lookup(idx, emb_weight)
    return out.reshape(BATCH, SEQLEN, EMB_DIM)


# speedup vs baseline: 1.5671x; 1.4010x over previous
import functools

import jax
import jax.numpy as jnp
from jax import lax
from jax.experimental import pallas as pl
from jax.experimental.pallas import tpu as pltpu
from jax.experimental.pallas import tpu_sc as plsc

N_VOCABS = 1000000
EMB_DIM = 64
BATCH = 4096
SEQLEN = 200

CHUNK = 128
TOTAL = BATCH * SEQLEN
NROWS = TOTAL // CHUNK
NW = 32
ROWS_PER_W = NROWS // NW
NBUF = 4
G = 2

VBLK = 4096


def _fmt_body(tT_ref, out_ref):
    xt = tT_ref[...].T
    out_ref[...] = jnp.concatenate([xt, xt], axis=-1)


@jax.jit
def _tc_format(tableT):
    grid = (N_VOCABS + VBLK - 1) // VBLK
    return pl.pallas_call(
        _fmt_body,
        grid=(grid,),
        in_specs=[pl.BlockSpec((EMB_DIM, VBLK), lambda i: (0, i))],
        out_specs=pl.BlockSpec((VBLK, 128), lambda i: (i, 0)),
        out_shape=jax.ShapeDtypeStruct((N_VOCABS, 128), jnp.float32),
    )(tableT)


def _emb_body(idx_hbm, table_hbm, out_hbm, idx_v, rows_v, *sems):
    gsem = sems[:NBUF]
    osem = sems[NBUF:]
    cid = lax.axis_index("c")
    sid = lax.axis_index("s")
    wid = sid * 2 + cid
    base = wid * ROWS_PER_W

    pltpu.sync_copy(idx_hbm.at[pl.ds(base, ROWS_PER_W)], idx_v)

    def start_gather(j, slot):
        pltpu.make_async_copy(
            table_hbm.at[idx_v.at[j]], rows_v.at[slot], gsem[slot]
        ).start()

    def wait_gather(j, slot):
        pltpu.make_async_copy(
            table_hbm.at[idx_v.at[j]], rows_v.at[slot], gsem[slot]
        ).wait()

    def start_write(j, slot):
        pltpu.make_async_copy(
            rows_v.at[slot], out_hbm.at[base + j], osem[slot]
        ).start()

    def wait_write(j, slot):
        pltpu.make_async_copy(
            rows_v.at[slot], out_hbm.at[base + j], osem[slot]
        ).wait()

    for j in range(G):
        start_gather(j, j % NBUF)
    for j in range(NBUF - G):
        wait_gather(j, j % NBUF)
        start_write(j, j % NBUF)
        start_gather(j + G, (j + G) % NBUF)

    @pl.loop(NBUF - G, ROWS_PER_W - G, step=NBUF)
    def _steady(j0):
        for b in range(NBUF):
            j = j0 + b
            slot = (NBUF - G + b) % NBUF
            sn = (NBUF - G + b + G) % NBUF
            wait_gather(j, slot)
            start_write(j, slot)
            wait_write(j + G - NBUF, sn)
            start_gather(j + G, sn)

    for j in range(ROWS_PER_W - G, ROWS_PER_W):
        wait_gather(j, j % NBUF)
        start_write(j, j % NBUF)
    for j in range(ROWS_PER_W - NBUF, ROWS_PER_W):
        wait_write(j, j % NBUF)


@jax.jit
def _emb_lookup(idx, table):
    k = pl.kernel(
        _emb_body,
        out_type=jax.ShapeDtypeStruct((NROWS, CHUNK, 128), jnp.float32),
        mesh=plsc.VectorSubcoreMesh(core_axis_name="c", subcore_axis_name="s"),
        compiler_params=pltpu.CompilerParams(use_tc_tiling_on_sc=True),
        scratch_types=(
            [
                pltpu.VMEM((ROWS_PER_W, CHUNK), jnp.int32),
                pltpu.VMEM((NBUF, CHUNK, 128), jnp.float32),
            ]
            + [pltpu.SemaphoreType.DMA] * (2 * NBUF)
        ),
    )
    return k(idx, table)


def kernel(input, emb_weight):
    table_pad = _tc_format(emb_weight.T)
    idx = input.reshape(NROWS, CHUNK)
    out = _emb_lookup(idx, table_pad)
    return out[:, :, :EMB_DIM].reshape(BATCH, SEQLEN, EMB_DIM)
